# baseline (speedup 1.0000x reference)
import functools

import jax
import jax.numpy as jnp
from jax import lax
from jax.experimental import pallas as pl
from jax.experimental.pallas import tpu as pltpu
from jax.experimental.pallas import tpu_sc as plsc

NC = 2
NS = 16
LANES = 16
NW = NC * NS

VPAD = 1024
CHUNK_ROWS = 8
NBUF = 2


def _build_counts_kernel(B, H):
    rows_per_w = B // NW
    n_chunks = rows_per_w // CHUNK_ROWS
    n_full = H // LANES
    tail = H - n_full * LANES
    n_vecs = n_full + (1 if tail else 0)
    mesh = plsc.VectorSubcoreMesh(
        core_axis_name="c", subcore_axis_name="s",
        num_cores=NC, num_subcores=NS)

    @functools.partial(
        pl.kernel,
        out_type=jax.ShapeDtypeStruct((B, VPAD), jnp.float32),
        mesh=mesh,
        scratch_types=[
            pltpu.VMEM((rows_per_w * H + LANES,), jnp.int32),
            pltpu.VMEM((CHUNK_ROWS, VPAD), jnp.float32),
            pltpu.VMEM((CHUNK_ROWS, VPAD), jnp.float32),
            pltpu.SemaphoreType.DMA,
            pltpu.SemaphoreType.DMA,
        ],
        compiler_params=pltpu.CompilerParams(needs_layout_passes=False),
    )
    def build_counts(s_hbm, c_hbm, s_v, hist0, hist1, sem0, sem1):
        wid = lax.axis_index("s") * NC + lax.axis_index("c")
        base = wid * rows_per_w
        pltpu.sync_copy(s_hbm.at[pl.ds(base * H, rows_per_w * H)],
                        s_v.at[pl.ds(0, rows_per_w * H)])
        sems = [sem0, sem1]
        hists = [hist0, hist1]

        zeros16 = jnp.zeros((LANES,), jnp.float32)
        ones16 = jnp.ones((LANES,), jnp.float32)
        tail_mask = lax.iota(jnp.int32, LANES) < tail

        def ship(k, buf):
            pltpu.async_copy(
                hists[buf],
                c_hbm.at[pl.ds(base + k * CHUNK_ROWS, CHUNK_ROWS)],
                sems[buf])

        def loop_body(k2, _):
            @pl.when(k2 > 0)
            def _wait():
                for buf in range(NBUF):
                    k = k2 * NBUF + buf
                    pltpu.make_async_copy(
                        hists[buf],
                        c_hbm.at[pl.ds(base + (k - NBUF) * CHUNK_ROWS,
                                       CHUNK_ROWS)],
                        sems[buf]).wait()
            for r in range(CHUNK_ROWS):
                for i in range(VPAD // LANES):
                    for buf in range(NBUF):
                        hists[buf][r, pl.ds(i * LANES, LANES)] = zeros16
            for r in range(CHUNK_ROWS):
                idxs = []
                for buf in range(NBUF):
                    row = (k2 * NBUF + buf) * CHUNK_ROWS + r
                    for j in range(n_vecs):
                        idxs.append(
                            (buf, j >= n_full,
                             s_v[pl.ds(row * H + j * LANES, LANES)]))
                row_splat = jnp.full((LANES,), r, jnp.int32)
                for buf, is_tail, idx in idxs:
                    plsc.addupdate_scatter(
                        hists[buf], [row_splat, idx], ones16,
                        mask=tail_mask if is_tail else None)
            for buf in range(NBUF):
                ship(k2 * NBUF + buf, buf)
            return 0

        lax.fori_loop(0, n_chunks // NBUF, loop_body, 0)
        for buf in range(NBUF):
            k = n_chunks - NBUF + buf
            pltpu.make_async_copy(
                hists[buf],
                c_hbm.at[pl.ds(base + k * CHUNK_ROWS, CHUNK_ROWS)],
                sems[buf]).wait()

    return build_counts


def _tc_body(hist_len, c_ref, a_ref, mu_ref, o_ref):
    c = c_ref[...]
    p = jnp.dot(c.astype(jnp.bfloat16), a_ref[...],
                preferred_element_type=jnp.float32)
    v = jnp.sum(p * p, axis=1, keepdims=True)
    m = jnp.dot(c, mu_ref[...], preferred_element_type=jnp.float32)
    o_ref[...] = jax.nn.sigmoid(m * lax.rsqrt(hist_len + v))


def kernel(S, mu, A):
    B, H = S.shape
    D = A.shape[1]
    counts = _build_counts_kernel(B, H)(S.astype(jnp.int32).reshape(B * H))

    a_bf = jnp.zeros((VPAD, D), jnp.bfloat16).at[:A.shape[0]].set(
        A.astype(jnp.bfloat16))
    mu2d = jnp.zeros((VPAD, 1), jnp.float32).at[:mu.shape[0], 0].set(mu)

    blk = 512
    out2d = pl.pallas_call(
        functools.partial(_tc_body, float(H)),
        grid=(B // blk,),
        in_specs=[
            pl.BlockSpec((blk, VPAD), lambda i: (i, 0)),
            pl.BlockSpec((VPAD, D), lambda i: (0, 0)),
            pl.BlockSpec((VPAD, 1), lambda i: (0, 0)),
        ],
        out_specs=pl.BlockSpec((blk, 1), lambda i: (i, 0)),
        out_shape=jax.ShapeDtypeStruct((B, 1), jnp.float32),
    )(counts, a_bf, mu2d)
    return out2d[:, 0]


# speedup vs baseline: 1.0915x; 1.0915x over previous
import functools

import jax
import jax.numpy as jnp
from jax import lax
from jax.experimental import pallas as pl
from jax.experimental.pallas import tpu as pltpu
from jax.experimental.pallas import tpu_sc as plsc

NC = 2
NS = 16
LANES = 16
NW = NC * NS

D_DIM = 1000
CHUNK_ROWS = 8
NBUF = 2


def _build_counts_kernel(B, H):
    rows_per_w = B // NW
    n_chunks = rows_per_w // CHUNK_ROWS
    n_full = H // LANES
    tail = H - n_full * LANES
    n_vecs = n_full + (1 if tail else 0)
    zero_offs = [i * LANES for i in range(D_DIM // LANES)]
    if D_DIM % LANES:
        zero_offs.append(D_DIM - LANES)
    mesh = plsc.VectorSubcoreMesh(
        core_axis_name="c", subcore_axis_name="s",
        num_cores=NC, num_subcores=NS)

    @functools.partial(
        pl.kernel,
        out_type=jax.ShapeDtypeStruct((B, D_DIM), jnp.float32),
        mesh=mesh,
        scratch_types=[
            pltpu.VMEM((rows_per_w, H), jnp.int32),
            pltpu.VMEM((CHUNK_ROWS, D_DIM), jnp.float32),
            pltpu.VMEM((CHUNK_ROWS, D_DIM), jnp.float32),
            pltpu.SemaphoreType.DMA,
            pltpu.SemaphoreType.DMA,
        ],
        compiler_params=pltpu.CompilerParams(needs_layout_passes=False),
    )
    def build_counts(s_hbm, c_hbm, s_v, hist0, hist1, sem0, sem1):
        wid = lax.axis_index("s") * NC + lax.axis_index("c")
        base = wid * rows_per_w
        pltpu.sync_copy(s_hbm.at[pl.ds(base, rows_per_w)], s_v)
        sems = [sem0, sem1]
        hists = [hist0, hist1]

        zeros16 = jnp.zeros((LANES,), jnp.float32)
        ones16 = jnp.ones((LANES,), jnp.float32)
        tail_mask = lax.iota(jnp.int32, LANES) >= (LANES - tail)

        def ship(k, buf):
            pltpu.async_copy(
                hists[buf],
                c_hbm.at[pl.ds(base + k * CHUNK_ROWS, CHUNK_ROWS)],
                sems[buf])

        def loop_body(k2, _):
            @pl.when(k2 > 0)
            def _wait():
                for buf in range(NBUF):
                    k = k2 * NBUF + buf
                    pltpu.make_async_copy(
                        hists[buf],
                        c_hbm.at[pl.ds(base + (k - NBUF) * CHUNK_ROWS,
                                       CHUNK_ROWS)],
                        sems[buf]).wait()
            for r in range(CHUNK_ROWS):
                for off in zero_offs:
                    for buf in range(NBUF):
                        hists[buf][r, pl.ds(off, LANES)] = zeros16
            for r in range(CHUNK_ROWS):
                idxs = []
                for buf in range(NBUF):
                    row = (k2 * NBUF + buf) * CHUNK_ROWS + r
                    for j in range(n_vecs):
                        col = j * LANES if j < n_full else H - LANES
                        idxs.append(
                            (buf, j >= n_full,
                             s_v[row, pl.ds(col, LANES)]))
                row_splat = jnp.full((LANES,), r, jnp.int32)
                for buf, is_tail, idx in idxs:
                    plsc.addupdate_scatter(
                        hists[buf], [row_splat, idx], ones16,
                        mask=tail_mask if is_tail else None)
            for buf in range(NBUF):
                ship(k2 * NBUF + buf, buf)
            return 0

        lax.fori_loop(0, n_chunks // NBUF, loop_body, 0)
        for buf in range(NBUF):
            k = n_chunks - NBUF + buf
            pltpu.make_async_copy(
                hists[buf],
                c_hbm.at[pl.ds(base + k * CHUNK_ROWS, CHUNK_ROWS)],
                sems[buf]).wait()

    return build_counts


def _tc_body(hist_len, c_ref, a_ref, mu_ref, o_ref):
    c = c_ref[...]
    p = jnp.dot(c.astype(jnp.bfloat16), a_ref[...],
                preferred_element_type=jnp.float32)
    v = jnp.sum(p * p, axis=1, keepdims=True)
    m = jnp.dot(c, mu_ref[...], preferred_element_type=jnp.float32)
    o_ref[...] = jax.nn.sigmoid(m * lax.rsqrt(hist_len + v))


def kernel(S, mu, A):
    B, H = S.shape
    D = A.shape[1]
    counts = _build_counts_kernel(B, H)(S.astype(jnp.int32))

    a_bf = A.astype(jnp.bfloat16)
    mu2d = mu.reshape(D, 1)

    blk = 512
    out2d = pl.pallas_call(
        functools.partial(_tc_body, float(H)),
        grid=(B // blk,),
        in_specs=[
            pl.BlockSpec((blk, D_DIM), lambda i: (i, 0)),
            pl.BlockSpec((D_DIM, D), lambda i: (0, 0)),
            pl.BlockSpec((D_DIM, 1), lambda i: (0, 0)),
        ],
        out_specs=pl.BlockSpec((blk, 1), lambda i: (i, 0)),
        out_shape=jax.ShapeDtypeStruct((B, 1), jnp.float32),
    )(counts, a_bf, mu2d)
    return out2d[:, 0]


# speedup vs baseline: 1.1062x; 1.0135x over previous
import functools

import jax
import jax.numpy as jnp
from jax import lax
from jax.experimental import pallas as pl
from jax.experimental.pallas import tpu as pltpu
from jax.experimental.pallas import tpu_sc as plsc

NC = 2
NS = 16
LANES = 16
NW = NC * NS

D_DIM = 1000
CHUNK_ROWS = 8
NBUF = 2


def _build_counts_kernel(B, H):
    rows_per_w = B // NW
    n_chunks = rows_per_w // CHUNK_ROWS
    n_full = H // LANES
    tail = H - n_full * LANES
    n_vecs = n_full + (1 if tail else 0)
    zero_offs = [i * LANES for i in range(D_DIM // LANES)]
    if D_DIM % LANES:
        zero_offs.append(D_DIM - LANES)
    mesh = plsc.VectorSubcoreMesh(
        core_axis_name="c", subcore_axis_name="s",
        num_cores=NC, num_subcores=NS)

    @functools.partial(
        pl.kernel,
        out_type=jax.ShapeDtypeStruct((B, D_DIM), jnp.float32),
        mesh=mesh,
        scratch_types=[
            pltpu.VMEM((rows_per_w, H), jnp.int32),
            pltpu.VMEM((CHUNK_ROWS, D_DIM), jnp.float32),
            pltpu.VMEM((CHUNK_ROWS, D_DIM), jnp.float32),
            pltpu.SemaphoreType.DMA,
            pltpu.SemaphoreType.DMA,
        ],
        compiler_params=pltpu.CompilerParams(
            needs_layout_passes=False, use_tc_tiling_on_sc=True),
    )
    def build_counts(s_hbm, c_hbm, s_v, hist0, hist1, sem0, sem1):
        wid = lax.axis_index("s") * NC + lax.axis_index("c")
        base = wid * rows_per_w
        pltpu.sync_copy(s_hbm.at[pl.ds(base, rows_per_w)], s_v)
        sems = [sem0, sem1]
        hists = [hist0, hist1]

        zeros16 = jnp.zeros((LANES,), jnp.float32)
        ones16 = jnp.ones((LANES,), jnp.float32)
        tail_mask = lax.iota(jnp.int32, LANES) >= (LANES - tail)

        def ship(k, buf):
            pltpu.async_copy(
                hists[buf],
                c_hbm.at[pl.ds(base + k * CHUNK_ROWS, CHUNK_ROWS)],
                sems[buf])

        def loop_body(k2, _):
            @pl.when(k2 > 0)
            def _wait():
                for buf in range(NBUF):
                    k = k2 * NBUF + buf
                    pltpu.make_async_copy(
                        hists[buf],
                        c_hbm.at[pl.ds(base + (k - NBUF) * CHUNK_ROWS,
                                       CHUNK_ROWS)],
                        sems[buf]).wait()
            for r in range(CHUNK_ROWS):
                for off in zero_offs:
                    for buf in range(NBUF):
                        hists[buf][r, pl.ds(off, LANES)] = zeros16
            for r in range(CHUNK_ROWS):
                idxs = []
                for buf in range(NBUF):
                    row = (k2 * NBUF + buf) * CHUNK_ROWS + r
                    for j in range(n_vecs):
                        col = j * LANES if j < n_full else H - LANES
                        idxs.append(
                            (buf, j >= n_full,
                             s_v[row, pl.ds(col, LANES)]))
                row_splat = jnp.full((LANES,), r, jnp.int32)
                for buf, is_tail, idx in idxs:
                    plsc.addupdate_scatter(
                        hists[buf], [row_splat, idx], ones16,
                        mask=tail_mask if is_tail else None)
            for buf in range(NBUF):
                ship(k2 * NBUF + buf, buf)
            return 0

        lax.fori_loop(0, n_chunks // NBUF, loop_body, 0)
        for buf in range(NBUF):
            k = n_chunks - NBUF + buf
            pltpu.make_async_copy(
                hists[buf],
                c_hbm.at[pl.ds(base + k * CHUNK_ROWS, CHUNK_ROWS)],
                sems[buf]).wait()

    return build_counts


def _tc_body(hist_len, c_ref, a_ref, mu_ref, o_ref):
    c = c_ref[...]
    p = jnp.dot(c.astype(jnp.bfloat16), a_ref[...],
                preferred_element_type=jnp.float32)
    v = jnp.sum(p * p, axis=1, keepdims=True)
    m = jnp.dot(c, mu_ref[...], preferred_element_type=jnp.float32)
    res = jax.nn.sigmoid(m * lax.rsqrt(hist_len + v))
    o_ref[...] = jnp.squeeze(res, -1)


def kernel(S, mu, A):
    B, H = S.shape
    D = A.shape[1]
    counts = _build_counts_kernel(B, H)(S.astype(jnp.int32))

    a_bf = A.astype(jnp.bfloat16)
    mu2d = mu.reshape(D, 1)

    blk = 512
    out2d = pl.pallas_call(
        functools.partial(_tc_body, float(H)),
        grid=(B // blk,),
        in_specs=[
            pl.BlockSpec((blk, D_DIM), lambda i: (i, 0)),
            pl.BlockSpec((D_DIM, D), lambda i: (0, 0)),
            pl.BlockSpec((D_DIM, 1), lambda i: (0, 0)),
        ],
        out_specs=pl.BlockSpec((blk,), lambda i: (i,)),
        out_shape=jax.ShapeDtypeStruct((B,), jnp.float32),
    )(counts, a_bf, mu2d)
    return out2d


# speedup vs baseline: 1.1084x; 1.0020x over previous
import functools

import jax
import jax.numpy as jnp
from jax import lax
from jax.experimental import pallas as pl
from jax.experimental.pallas import tpu as pltpu
from jax.experimental.pallas import tpu_sc as plsc

NC = 2
NS = 16
LANES = 16
NW = NC * NS

D_DIM = 1000
CHUNK_ROWS = 8
NBUF = 2


def _build_counts_kernel(B, H):
    rows_per_w = B // NW
    n_chunks = rows_per_w // CHUNK_ROWS
    n_full = H // LANES
    tail = H - n_full * LANES
    n_vecs = n_full + (1 if tail else 0)
    zero_offs = [i * LANES for i in range(D_DIM // LANES)]
    if D_DIM % LANES:
        zero_offs.append(D_DIM - LANES)
    mesh = plsc.VectorSubcoreMesh(
        core_axis_name="c", subcore_axis_name="s",
        num_cores=NC, num_subcores=NS)

    @functools.partial(
        pl.kernel,
        out_type=jax.ShapeDtypeStruct((B, D_DIM), jnp.float32),
        mesh=mesh,
        scratch_types=[
            pltpu.VMEM((rows_per_w, H), jnp.int32),
            pltpu.VMEM((CHUNK_ROWS, D_DIM), jnp.float32),
            pltpu.VMEM((CHUNK_ROWS, D_DIM), jnp.float32),
            pltpu.SemaphoreType.DMA,
            pltpu.SemaphoreType.DMA,
        ],
        compiler_params=pltpu.CompilerParams(
            needs_layout_passes=False, use_tc_tiling_on_sc=True),
    )
    def build_counts(s_hbm, c_hbm, s_v, hist0, hist1, sem0, sem1):
        wid = lax.axis_index("s") * NC + lax.axis_index("c")
        base = wid * rows_per_w
        pltpu.sync_copy(s_hbm.at[pl.ds(base, rows_per_w)], s_v)
        sems = [sem0, sem1]
        hists = [hist0, hist1]

        zeros16 = jnp.zeros((LANES,), jnp.float32)
        ones16 = jnp.ones((LANES,), jnp.float32)
        tail_mask = lax.iota(jnp.int32, LANES) >= (LANES - tail)

        def ship(k, buf):
            pltpu.async_copy(
                hists[buf],
                c_hbm.at[pl.ds(base + k * CHUNK_ROWS, CHUNK_ROWS)],
                sems[buf])

        def loop_body(k2, _):
            @pl.when(k2 > 0)
            def _wait():
                for buf in range(NBUF):
                    k = k2 * NBUF + buf
                    pltpu.make_async_copy(
                        hists[buf],
                        c_hbm.at[pl.ds(base + (k - NBUF) * CHUNK_ROWS,
                                       CHUNK_ROWS)],
                        sems[buf]).wait()
            for r in range(CHUNK_ROWS):
                for off in zero_offs:
                    for buf in range(NBUF):
                        hists[buf][r, pl.ds(off, LANES)] = zeros16
            for r in range(CHUNK_ROWS):
                idxs = []
                for buf in range(NBUF):
                    row = (k2 * NBUF + buf) * CHUNK_ROWS + r
                    for j in range(n_vecs):
                        col = j * LANES if j < n_full else H - LANES
                        idxs.append(
                            (buf, j >= n_full,
                             s_v[row, pl.ds(col, LANES)]))
                row_splat = jnp.full((LANES,), r, jnp.int32)
                for buf, is_tail, idx in idxs:
                    plsc.addupdate_scatter(
                        hists[buf], [row_splat, idx], ones16,
                        mask=tail_mask if is_tail else None)
            for buf in range(NBUF):
                ship(k2 * NBUF + buf, buf)
            return 0

        lax.fori_loop(0, n_chunks // NBUF, loop_body, 0)
        for buf in range(NBUF):
            k = n_chunks - NBUF + buf
            pltpu.make_async_copy(
                hists[buf],
                c_hbm.at[pl.ds(base + k * CHUNK_ROWS, CHUNK_ROWS)],
                sems[buf]).wait()

    return build_counts


def _tc_body(hist_len, c_ref, a_ref, mu_ref, o_ref):
    c = c_ref[...]
    p = jnp.dot(c, a_ref[...], preferred_element_type=jnp.float32)
    v = jnp.sum(p * p, axis=1, keepdims=True)
    m = jnp.dot(c, mu_ref[...].reshape(-1, 1),
                preferred_element_type=jnp.float32)
    res = jax.nn.sigmoid(m * lax.rsqrt(hist_len + v))
    o_ref[...] = jnp.squeeze(res, -1)


def _half(S, mu, A, H, D):
    b = S.shape[0]
    counts = _build_counts_kernel(b, H)(S)
    blk = 512
    return pl.pallas_call(
        functools.partial(_tc_body, float(H)),
        grid=(b // blk,),
        in_specs=[
            pl.BlockSpec((blk, D_DIM), lambda i: (i, 0)),
            pl.BlockSpec((D, D), lambda i: (0, 0)),
            pl.BlockSpec((D,), lambda i: (0,)),
        ],
        out_specs=pl.BlockSpec((blk,), lambda i: (i,)),
        out_shape=jax.ShapeDtypeStruct((b,), jnp.float32),
    )(counts, A, mu)


def kernel(S, mu, A):
    B, H = S.shape
    D = A.shape[1]
    s32 = S.astype(jnp.int32)
    o0 = _half(s32[:B // 2], mu, A, H, D)
    o1 = _half(s32[B // 2:], mu, A, H, D)
    return jnp.concatenate([o0, o1])


# speedup vs baseline: 1.2709x; 1.1466x over previous
import functools

import jax
import jax.numpy as jnp
from jax import lax
from jax.experimental import pallas as pl
from jax.experimental.pallas import tpu as pltpu
from jax.experimental.pallas import tpu_sc as plsc

NC = 2
NS = 16
LANES = 16
NW = NC * NS

D_DIM = 1000
CHUNK_ROWS = 4
NBUF = 2


def _build_counts_kernel(B, H):
    rows_per_w = B // NW
    n_chunks = rows_per_w // CHUNK_ROWS
    n_full = H // LANES
    tail = H - n_full * LANES
    n_vecs = n_full + (1 if tail else 0)
    zero_offs = [i * LANES for i in range(D_DIM // LANES)]
    if D_DIM % LANES:
        zero_offs.append(D_DIM - LANES)
    mesh = plsc.VectorSubcoreMesh(
        core_axis_name="c", subcore_axis_name="s",
        num_cores=NC, num_subcores=NS)

    @functools.partial(
        pl.kernel,
        out_type=jax.ShapeDtypeStruct((B, D_DIM), jnp.float32),
        mesh=mesh,
        scratch_types=[
            pltpu.VMEM((rows_per_w, H), jnp.int32),
            pltpu.VMEM((CHUNK_ROWS, D_DIM), jnp.float32),
            pltpu.VMEM((CHUNK_ROWS, D_DIM), jnp.float32),
            pltpu.SemaphoreType.DMA,
            pltpu.SemaphoreType.DMA,
        ],
        compiler_params=pltpu.CompilerParams(
            needs_layout_passes=False, use_tc_tiling_on_sc=True),
    )
    def build_counts(s_hbm, c_hbm, s_v, hist0, hist1, sem0, sem1):
        wid = lax.axis_index("s") * NC + lax.axis_index("c")
        base = wid * rows_per_w
        pltpu.sync_copy(s_hbm.at[pl.ds(base, rows_per_w)], s_v)
        sems = [sem0, sem1]
        hists = [hist0, hist1]

        zeros16 = jnp.zeros((LANES,), jnp.float32)
        ones16 = jnp.ones((LANES,), jnp.float32)
        tail_mask = lax.iota(jnp.int32, LANES) >= (LANES - tail)

        def ship(k, buf):
            pltpu.async_copy(
                hists[buf],
                c_hbm.at[pl.ds(base + k * CHUNK_ROWS, CHUNK_ROWS)],
                sems[buf])

        def loop_body(k2, _):
            @pl.when(k2 > 0)
            def _wait():
                for buf in range(NBUF):
                    k = k2 * NBUF + buf
                    pltpu.make_async_copy(
                        hists[buf],
                        c_hbm.at[pl.ds(base + (k - NBUF) * CHUNK_ROWS,
                                       CHUNK_ROWS)],
                        sems[buf]).wait()
            for r in range(CHUNK_ROWS):
                for off in zero_offs:
                    for buf in range(NBUF):
                        hists[buf][r, pl.ds(off, LANES)] = zeros16
            for r in range(CHUNK_ROWS):
                idxs = []
                for buf in range(NBUF):
                    row = (k2 * NBUF + buf) * CHUNK_ROWS + r
                    for j in range(n_vecs):
                        col = j * LANES if j < n_full else H - LANES
                        idxs.append(
                            (buf, j >= n_full,
                             s_v[row, pl.ds(col, LANES)]))
                row_splat = jnp.full((LANES,), r, jnp.int32)
                for buf, is_tail, idx in idxs:
                    plsc.addupdate_scatter(
                        hists[buf], [row_splat, idx], ones16,
                        mask=tail_mask if is_tail else None)
            for buf in range(NBUF):
                ship(k2 * NBUF + buf, buf)
            return 0

        lax.fori_loop(0, n_chunks // NBUF, loop_body, 0)
        for buf in range(NBUF):
            k = n_chunks - NBUF + buf
            pltpu.make_async_copy(
                hists[buf],
                c_hbm.at[pl.ds(base + k * CHUNK_ROWS, CHUNK_ROWS)],
                sems[buf]).wait()

    return build_counts


def _tc_body(hist_len, c_ref, a_ref, mu_ref, o_ref):
    c = c_ref[...]
    p = jnp.dot(c, a_ref[...], preferred_element_type=jnp.float32)
    v = jnp.sum(p * p, axis=1, keepdims=True)
    m = jnp.dot(c, mu_ref[...].reshape(-1, 1),
                preferred_element_type=jnp.float32)
    res = jax.nn.sigmoid(m * lax.rsqrt(hist_len + v))
    o_ref[...] = jnp.squeeze(res, -1)


def _half(S, mu, A, H, D):
    b = S.shape[0]
    counts = _build_counts_kernel(b, H)(S)
    blk = 512
    return pl.pallas_call(
        functools.partial(_tc_body, float(H)),
        grid=(b // blk,),
        in_specs=[
            pl.BlockSpec((blk, D_DIM), lambda i: (i, 0)),
            pl.BlockSpec((D, D), lambda i: (0, 0)),
            pl.BlockSpec((D,), lambda i: (0,)),
        ],
        out_specs=pl.BlockSpec((blk,), lambda i: (i,)),
        out_shape=jax.ShapeDtypeStruct((b,), jnp.float32),
    )(counts, A, mu)


def kernel(S, mu, A):
    B, H = S.shape
    D = A.shape[1]
    s32 = S.astype(jnp.int32)
    o0 = _half(s32[:B // 2], mu, A, H, D)
    o1 = _half(s32[B // 2:], mu, A, H, D)
    return jnp.concatenate([o0, o1])


# speedup vs baseline: 1.2750x; 1.0032x over previous
import functools

import jax
import jax.numpy as jnp
from jax import lax
from jax.experimental import pallas as pl
from jax.experimental.pallas import tpu as pltpu
from jax.experimental.pallas import tpu_sc as plsc

NC = 2
NS = 16
LANES = 16
NW = NC * NS

D_DIM = 1000
CHUNK_ROWS = 2
NBUF = 2


def _build_counts_kernel(B, H):
    rows_per_w = B // NW
    n_chunks = rows_per_w // CHUNK_ROWS
    n_full = H // LANES
    tail = H - n_full * LANES
    n_vecs = n_full + (1 if tail else 0)
    zero_offs = [i * LANES for i in range(D_DIM // LANES)]
    if D_DIM % LANES:
        zero_offs.append(D_DIM - LANES)
    mesh = plsc.VectorSubcoreMesh(
        core_axis_name="c", subcore_axis_name="s",
        num_cores=NC, num_subcores=NS)

    @functools.partial(
        pl.kernel,
        out_type=jax.ShapeDtypeStruct((B, D_DIM), jnp.float32),
        mesh=mesh,
        scratch_types=[
            pltpu.VMEM((rows_per_w, H), jnp.int32),
            pltpu.VMEM((CHUNK_ROWS, D_DIM), jnp.float32),
            pltpu.VMEM((CHUNK_ROWS, D_DIM), jnp.float32),
            pltpu.SemaphoreType.DMA,
            pltpu.SemaphoreType.DMA,
        ],
        compiler_params=pltpu.CompilerParams(
            needs_layout_passes=False, use_tc_tiling_on_sc=True),
    )
    def build_counts(s_hbm, c_hbm, s_v, hist0, hist1, sem0, sem1):
        wid = lax.axis_index("s") * NC + lax.axis_index("c")
        base = wid * rows_per_w
        pltpu.sync_copy(s_hbm.at[pl.ds(base, rows_per_w)], s_v)
        sems = [sem0, sem1]
        hists = [hist0, hist1]

        zeros16 = jnp.zeros((LANES,), jnp.float32)
        ones16 = jnp.ones((LANES,), jnp.float32)
        tail_mask = lax.iota(jnp.int32, LANES) >= (LANES - tail)

        def ship(k, buf):
            pltpu.async_copy(
                hists[buf],
                c_hbm.at[pl.ds(base + k * CHUNK_ROWS, CHUNK_ROWS)],
                sems[buf])

        def loop_body(k2, _):
            @pl.when(k2 > 0)
            def _wait():
                for buf in range(NBUF):
                    k = k2 * NBUF + buf
                    pltpu.make_async_copy(
                        hists[buf],
                        c_hbm.at[pl.ds(base + (k - NBUF) * CHUNK_ROWS,
                                       CHUNK_ROWS)],
                        sems[buf]).wait()
            for r in range(CHUNK_ROWS):
                for off in zero_offs:
                    for buf in range(NBUF):
                        hists[buf][r, pl.ds(off, LANES)] = zeros16
            for r in range(CHUNK_ROWS):
                idxs = []
                for buf in range(NBUF):
                    row = (k2 * NBUF + buf) * CHUNK_ROWS + r
                    for j in range(n_vecs):
                        col = j * LANES if j < n_full else H - LANES
                        idxs.append(
                            (buf, j >= n_full,
                             s_v[row, pl.ds(col, LANES)]))
                row_splat = jnp.full((LANES,), r, jnp.int32)
                for buf, is_tail, idx in idxs:
                    plsc.addupdate_scatter(
                        hists[buf], [row_splat, idx], ones16,
                        mask=tail_mask if is_tail else None)
            for buf in range(NBUF):
                ship(k2 * NBUF + buf, buf)
            return 0

        lax.fori_loop(0, n_chunks // NBUF, loop_body, 0)
        for buf in range(NBUF):
            k = n_chunks - NBUF + buf
            pltpu.make_async_copy(
                hists[buf],
                c_hbm.at[pl.ds(base + k * CHUNK_ROWS, CHUNK_ROWS)],
                sems[buf]).wait()

    return build_counts


def _tc_body(hist_len, c_ref, a_ref, mu_ref, o_ref):
    c = c_ref[...]
    p = jnp.dot(c, a_ref[...], preferred_element_type=jnp.float32)
    v = jnp.sum(p * p, axis=1, keepdims=True)
    m = jnp.dot(c, mu_ref[...].reshape(-1, 1),
                preferred_element_type=jnp.float32)
    res = jax.nn.sigmoid(m * lax.rsqrt(hist_len + v))
    o_ref[...] = jnp.squeeze(res, -1)


def _half(S, mu, A, H, D):
    b = S.shape[0]
    counts = _build_counts_kernel(b, H)(S)
    blk = 512
    return pl.pallas_call(
        functools.partial(_tc_body, float(H)),
        grid=(b // blk,),
        in_specs=[
            pl.BlockSpec((blk, D_DIM), lambda i: (i, 0)),
            pl.BlockSpec((D, D), lambda i: (0, 0)),
            pl.BlockSpec((D,), lambda i: (0,)),
        ],
        out_specs=pl.BlockSpec((blk,), lambda i: (i,)),
        out_shape=jax.ShapeDtypeStruct((b,), jnp.float32),
    )(counts, A, mu)


def kernel(S, mu, A):
    B, H = S.shape
    D = A.shape[1]
    s32 = S.astype(jnp.int32)
    o0 = _half(s32[:B // 2], mu, A, H, D)
    o1 = _half(s32[B // 2:], mu, A, H, D)
    return jnp.concatenate([o0, o1])
